# baseline (speedup 1.0000x reference)
import functools

import jax
import jax.numpy as jnp
from jax import lax
from jax.experimental import pallas as pl
from jax.experimental.pallas import tpu as pltpu
from jax.experimental.pallas import tpu_sc as plsc

N = 10000
E = 320000
D_IN = 128
D_H = 256
N_CLS = 2
BN_EPS = 1e-5

NC = 2
NS = 16
K = 128
N_PAD = 10240
CHUNKS = 160
E_PAD = NS * CHUNKS * K
GI = 16


def _zero_acc(rows0, acc, sid, rows_pt, dh):
    zero = jnp.zeros((16,), jnp.float32)

    def zrow(i, carry):
        for kk in range(dh // 16):
            rows0[i, pl.ds(kk * 16, 16)] = zero
        return carry

    lax.fori_loop(0, K, zrow, 0)
    rbase = sid * rows_pt
    for b in range(rows_pt // K):
        pltpu.sync_copy(rows0, acc.at[pl.ds(rbase + b * K, K)])


def _edge_pipeline(x_hbm, src_hbm, dst_hbm, tbase, n_groups,
                   src_v, dst_v, rows, gsems, ssems, acc):
    def group(g, carry):
        gb = tbase + g * GI
        pltpu.sync_copy(src_hbm.at[pl.ds(gb, GI)], src_v)
        pltpu.sync_copy(dst_hbm.at[pl.ds(gb, GI)], dst_v)
        gdesc = [None, None]
        gdesc[0] = pltpu.async_copy(x_hbm.at[src_v.at[0]], rows[0], gsems[0])
        for j in range(GI):
            b = j & 1
            nb = 1 - b
            if j + 1 < GI:
                gdesc[nb] = pltpu.async_copy(
                    x_hbm.at[src_v.at[j + 1]], rows[nb], gsems[nb])
            gdesc[b].wait()
            pltpu.sync_copy(rows[b], acc.at[dst_v.at[j]], add=True)
        return carry

    lax.fori_loop(0, n_groups, group, 0)


def _sc_aggr_edgesplit():
    rows_pt = N_PAD // NS
    ch = E_PAD // (NC * NS * K)
    mesh = plsc.VectorSubcoreMesh(
        core_axis_name="c", subcore_axis_name="s",
        num_cores=NC, num_subcores=NS)

    @functools.partial(
        pl.kernel,
        out_type=jax.ShapeDtypeStruct((2, N_PAD, D_IN), jnp.float32),
        mesh=mesh,
        scratch_types=[
            pltpu.VMEM((GI, K), jnp.int32),
            pltpu.VMEM((GI, K), jnp.int32),
            pltpu.VMEM((K, D_IN), jnp.float32),
            pltpu.VMEM((K, D_IN), jnp.float32),
            pltpu.VMEM_SHARED((N_PAD, D_IN), jnp.float32),
            pltpu.SemaphoreType.DMA,
            pltpu.SemaphoreType.DMA,
            pltpu.SemaphoreType.DMA,
            pltpu.SemaphoreType.DMA,
        ],
    )
    def aggr(x_hbm, src_hbm, dst_hbm, out,
             src_v, dst_v, rows0, rows1, acc, g0, g1, s0, s1):
        cid = lax.axis_index("c")
        sid = lax.axis_index("s")
        tbase = (cid * NS + sid) * ch
        _zero_acc(rows0, acc, sid, rows_pt, D_IN)
        plsc.subcore_barrier()
        _edge_pipeline(x_hbm, src_hbm, dst_hbm, tbase, ch // GI,
                       src_v, dst_v, [rows0, rows1], [g0, g1], [s0, s1], acc)
        plsc.subcore_barrier()
        rbase = sid * rows_pt
        pltpu.sync_copy(acc.at[pl.ds(rbase, rows_pt)],
                        out.at[cid, pl.ds(rbase, rows_pt)])

    return aggr


def _sc_aggr_colsplit():
    dh = 128
    rows_pt = N_PAD // NS
    mesh = plsc.VectorSubcoreMesh(
        core_axis_name="c", subcore_axis_name="s",
        num_cores=NC, num_subcores=NS)

    @functools.partial(
        pl.kernel,
        out_type=jax.ShapeDtypeStruct((2, N_PAD, dh), jnp.float32),
        mesh=mesh,
        scratch_types=[
            pltpu.VMEM((GI, K), jnp.int32),
            pltpu.VMEM((GI, K), jnp.int32),
            pltpu.VMEM((K, dh), jnp.float32),
            pltpu.VMEM((K, dh), jnp.float32),
            pltpu.VMEM_SHARED((N_PAD, dh), jnp.float32),
            pltpu.SemaphoreType.DMA,
            pltpu.SemaphoreType.DMA,
            pltpu.SemaphoreType.DMA,
            pltpu.SemaphoreType.DMA,
        ],
    )
    def aggr(xlo, xhi, src_hbm, dst_hbm, out,
             src_v, dst_v, rows0, rows1, acc, g0, g1, s0, s1):
        cid = lax.axis_index("c")
        sid = lax.axis_index("s")
        tbase = sid * CHUNKS
        _zero_acc(rows0, acc, sid, rows_pt, dh)
        plsc.subcore_barrier()

        @pl.when(cid == 0)
        def _():
            _edge_pipeline(xlo, src_hbm, dst_hbm, tbase, CHUNKS // GI,
                           src_v, dst_v, [rows0, rows1], [g0, g1], [s0, s1],
                           acc)

        @pl.when(cid == 1)
        def _():
            _edge_pipeline(xhi, src_hbm, dst_hbm, tbase, CHUNKS // GI,
                           src_v, dst_v, [rows0, rows1], [g0, g1], [s0, s1],
                           acc)

        plsc.subcore_barrier()
        rbase = sid * rows_pt
        pltpu.sync_copy(acc.at[pl.ds(rbase, rows_pt)],
                        out.at[cid, pl.ds(rbase, rows_pt)])

    return aggr


def _tc_layer1(br):
    grid = N_PAD // br

    def body(x_ref, agga_ref, aggb_ref, w_ref, b_ref, lo_ref, hi_ref):
        z = x_ref[...] + agga_ref[0] + aggb_ref[0]
        h = jnp.dot(z, w_ref[...], preferred_element_type=jnp.float32)
        h = jnp.maximum(h + b_ref[...], 0.0)
        lo_ref[...] = h[:, :128]
        hi_ref[...] = h[:, 128:]

    return pl.pallas_call(
        body,
        grid=(grid,),
        in_specs=[
            pl.BlockSpec((br, D_IN), lambda i: (i, 0)),
            pl.BlockSpec((1, br, D_IN), lambda i: (0, i, 0)),
            pl.BlockSpec((1, br, D_IN), lambda i: (1, i, 0)),
            pl.BlockSpec((D_IN, D_H), lambda i: (0, 0)),
            pl.BlockSpec((1, D_H), lambda i: (0, 0)),
        ],
        out_specs=[
            pl.BlockSpec((br, 128), lambda i: (i, 0)),
            pl.BlockSpec((br, 128), lambda i: (i, 0)),
        ],
        out_shape=[jax.ShapeDtypeStruct((N_PAD, 128), jnp.float32)] * 2,
    )


def _tc_layer2(br):
    grid = N_PAD // br

    def body(lo_ref, hi_ref, agglo_ref, agghi_ref, w_ref, b_ref, wc_ref, bc_ref,
             out_ref):
        h1 = jnp.concatenate([lo_ref[...], hi_ref[...]], axis=1)
        agg = jnp.concatenate([agglo_ref[0], agghi_ref[0]], axis=1)
        z = h1 + agg
        h = jnp.dot(z, w_ref[...], preferred_element_type=jnp.float32)
        h = jnp.maximum(h + b_ref[...], 0.0)
        out_ref[...] = (jnp.dot(h, wc_ref[...], preferred_element_type=jnp.float32)
                        + bc_ref[...])

    return pl.pallas_call(
        body,
        grid=(grid,),
        in_specs=[
            pl.BlockSpec((br, 128), lambda i: (i, 0)),
            pl.BlockSpec((br, 128), lambda i: (i, 0)),
            pl.BlockSpec((1, br, 128), lambda i: (0, i, 0)),
            pl.BlockSpec((1, br, 128), lambda i: (1, i, 0)),
            pl.BlockSpec((D_H, D_H), lambda i: (0, 0)),
            pl.BlockSpec((1, D_H), lambda i: (0, 0)),
            pl.BlockSpec((D_H, N_CLS), lambda i: (0, 0)),
            pl.BlockSpec((1, N_CLS), lambda i: (0, 0)),
        ],
        out_specs=pl.BlockSpec((br, N_CLS), lambda i: (i, 0)),
        out_shape=jax.ShapeDtypeStruct((N_PAD, N_CLS), jnp.float32),
    )


def kernel(x, edge_index, W1, b1, g1, be1, W2, b2, g2, be2, Wc, bc):
    src = edge_index[0]
    dst = edge_index[1]
    pad = E_PAD - E
    padv_src = jnp.full((pad,), N, jnp.int32)
    padv_dst = N + (jnp.arange(pad, dtype=jnp.int32) % (N_PAD - N))
    srcp = jnp.concatenate([src, padv_src]).reshape(-1, K)
    dstp = jnp.concatenate([dst, padv_dst]).reshape(-1, K)
    x_pad = jnp.pad(x, ((0, N_PAD - N), (0, 0)))

    s1 = g1 / jnp.sqrt(1.0 + BN_EPS)
    W1f = W1 * s1[None, :]
    b1f = (b1 * s1 + be1)[None, :]
    s2 = g2 / jnp.sqrt(1.0 + BN_EPS)
    W2f = W2 * s2[None, :]
    b2f = (b2 * s2 + be2)[None, :]

    agg1 = _sc_aggr_edgesplit()(x_pad, srcp, dstp)
    h1lo, h1hi = _tc_layer1(1024)(x_pad, agg1, agg1, W1f, b1f)
    agg2 = _sc_aggr_colsplit()(h1lo, h1hi, srcp, dstp)
    out = _tc_layer2(1024)(h1lo, h1hi, agg2, agg2, W2f, b2f, Wc, bc[None, :])
    return out[:N]


# speedup vs baseline: 3.1523x; 3.1523x over previous
import functools

import jax
import jax.numpy as jnp
from jax import lax
from jax.experimental import pallas as pl
from jax.experimental.pallas import tpu as pltpu
from jax.experimental.pallas import tpu_sc as plsc

N = 10000
E = 320000
D_IN = 128
D_H = 256
N_CLS = 2
BN_EPS = 1e-5

NC = 2
NS = 16
K = 128
N_PAD = 10240
CHUNKS = 160
E_PAD = NS * CHUNKS * K
GI = 40


def _zero_acc(rows0, acc, sid, rows_pt, dh):
    zero = jnp.zeros((16,), jnp.float32)

    def zrow(i, carry):
        for kk in range(dh // 16):
            rows0[i, pl.ds(kk * 16, 16)] = zero
        return carry

    lax.fori_loop(0, K, zrow, 0)
    rbase = sid * rows_pt
    for b in range(rows_pt // K):
        pltpu.sync_copy(rows0, acc.at[pl.ds(rbase + b * K, K)])


def _edge_pipeline(x_hbm, src_hbm, dst_hbm, tbase, n_groups,
                   src_v, dst_v, rows, gsems, ssems, acc):
    def group(g, carry):
        gb = tbase + g * GI
        pltpu.sync_copy(src_hbm.at[pl.ds(gb, GI)], src_v)
        pltpu.sync_copy(dst_hbm.at[pl.ds(gb, GI)], dst_v)
        gdesc = [None, None]
        gdesc[0] = pltpu.async_copy(x_hbm.at[src_v.at[0]], rows[0], gsems[0])
        for j in range(GI):
            b = j & 1
            nb = 1 - b
            if j + 1 < GI:
                gdesc[nb] = pltpu.async_copy(
                    x_hbm.at[src_v.at[j + 1]], rows[nb], gsems[nb])
            gdesc[b].wait()
            pltpu.sync_copy(rows[b], acc.at[dst_v.at[j]], add=True)
        return carry

    lax.fori_loop(0, n_groups, group, 0)


def _sc_aggr_edgesplit():
    rows_pt = N_PAD // NS
    ch = E_PAD // (NC * NS * K)
    mesh = plsc.VectorSubcoreMesh(
        core_axis_name="c", subcore_axis_name="s",
        num_cores=NC, num_subcores=NS)

    @functools.partial(
        pl.kernel,
        out_type=jax.ShapeDtypeStruct((2, N_PAD, D_IN), jnp.float32),
        mesh=mesh,
        scratch_types=[
            pltpu.VMEM((GI, K), jnp.int32),
            pltpu.VMEM((GI, K), jnp.int32),
            pltpu.VMEM((K, D_IN), jnp.float32),
            pltpu.VMEM((K, D_IN), jnp.float32),
            pltpu.VMEM_SHARED((N_PAD, D_IN), jnp.float32),
            pltpu.SemaphoreType.DMA,
            pltpu.SemaphoreType.DMA,
            pltpu.SemaphoreType.DMA,
            pltpu.SemaphoreType.DMA,
        ],
    )
    def aggr(x_hbm, src_hbm, dst_hbm, out,
             src_v, dst_v, rows0, rows1, acc, g0, g1, s0, s1):
        cid = lax.axis_index("c")
        sid = lax.axis_index("s")
        tbase = (cid * NS + sid) * ch
        _zero_acc(rows0, acc, sid, rows_pt, D_IN)
        plsc.subcore_barrier()
        _edge_pipeline(x_hbm, src_hbm, dst_hbm, tbase, ch // GI,
                       src_v, dst_v, [rows0, rows1], [g0, g1], [s0, s1], acc)
        plsc.subcore_barrier()
        rbase = sid * rows_pt
        pltpu.sync_copy(acc.at[pl.ds(rbase, rows_pt)],
                        out.at[cid, pl.ds(rbase, rows_pt)])

    return aggr


def _sc_aggr_colsplit():
    dh = 128
    rows_pt = N_PAD // NS
    mesh = plsc.VectorSubcoreMesh(
        core_axis_name="c", subcore_axis_name="s",
        num_cores=NC, num_subcores=NS)

    @functools.partial(
        pl.kernel,
        out_type=jax.ShapeDtypeStruct((2, N_PAD, dh), jnp.float32),
        mesh=mesh,
        scratch_types=[
            pltpu.VMEM((GI, K), jnp.int32),
            pltpu.VMEM((GI, K), jnp.int32),
            pltpu.VMEM((K, dh), jnp.float32),
            pltpu.VMEM((K, dh), jnp.float32),
            pltpu.VMEM_SHARED((N_PAD, dh), jnp.float32),
            pltpu.SemaphoreType.DMA,
            pltpu.SemaphoreType.DMA,
            pltpu.SemaphoreType.DMA,
            pltpu.SemaphoreType.DMA,
        ],
    )
    def aggr(xlo, xhi, src_hbm, dst_hbm, out,
             src_v, dst_v, rows0, rows1, acc, g0, g1, s0, s1):
        cid = lax.axis_index("c")
        sid = lax.axis_index("s")
        tbase = sid * CHUNKS
        _zero_acc(rows0, acc, sid, rows_pt, dh)
        plsc.subcore_barrier()

        @pl.when(cid == 0)
        def _():
            _edge_pipeline(xlo, src_hbm, dst_hbm, tbase, CHUNKS // GI,
                           src_v, dst_v, [rows0, rows1], [g0, g1], [s0, s1],
                           acc)

        @pl.when(cid == 1)
        def _():
            _edge_pipeline(xhi, src_hbm, dst_hbm, tbase, CHUNKS // GI,
                           src_v, dst_v, [rows0, rows1], [g0, g1], [s0, s1],
                           acc)

        plsc.subcore_barrier()
        rbase = sid * rows_pt
        pltpu.sync_copy(acc.at[pl.ds(rbase, rows_pt)],
                        out.at[cid, pl.ds(rbase, rows_pt)])

    return aggr


def _tc_layer1(br):
    grid = N_PAD // br

    def body(x_ref, agga_ref, aggb_ref, w_ref, b_ref, lo_ref, hi_ref):
        z = x_ref[...] + agga_ref[0] + aggb_ref[0]
        h = jnp.dot(z, w_ref[...], preferred_element_type=jnp.float32)
        h = jnp.maximum(h + b_ref[...], 0.0)
        lo_ref[...] = h[:, :128]
        hi_ref[...] = h[:, 128:]

    return pl.pallas_call(
        body,
        grid=(grid,),
        in_specs=[
            pl.BlockSpec((br, D_IN), lambda i: (i, 0)),
            pl.BlockSpec((1, br, D_IN), lambda i: (0, i, 0)),
            pl.BlockSpec((1, br, D_IN), lambda i: (1, i, 0)),
            pl.BlockSpec((D_IN, D_H), lambda i: (0, 0)),
            pl.BlockSpec((1, D_H), lambda i: (0, 0)),
        ],
        out_specs=[
            pl.BlockSpec((br, 128), lambda i: (i, 0)),
            pl.BlockSpec((br, 128), lambda i: (i, 0)),
        ],
        out_shape=[jax.ShapeDtypeStruct((N_PAD, 128), jnp.float32)] * 2,
    )


def _tc_layer2(br):
    grid = N_PAD // br

    def body(lo_ref, hi_ref, agglo_ref, agghi_ref, w_ref, b_ref, wc_ref, bc_ref,
             out_ref):
        h1 = jnp.concatenate([lo_ref[...], hi_ref[...]], axis=1)
        agg = jnp.concatenate([agglo_ref[0], agghi_ref[0]], axis=1)
        z = h1 + agg
        h = jnp.dot(z, w_ref[...], preferred_element_type=jnp.float32)
        h = jnp.maximum(h + b_ref[...], 0.0)
        out_ref[...] = (jnp.dot(h, wc_ref[...], preferred_element_type=jnp.float32)
                        + bc_ref[...])

    return pl.pallas_call(
        body,
        grid=(grid,),
        in_specs=[
            pl.BlockSpec((br, 128), lambda i: (i, 0)),
            pl.BlockSpec((br, 128), lambda i: (i, 0)),
            pl.BlockSpec((1, br, 128), lambda i: (0, i, 0)),
            pl.BlockSpec((1, br, 128), lambda i: (1, i, 0)),
            pl.BlockSpec((D_H, D_H), lambda i: (0, 0)),
            pl.BlockSpec((1, D_H), lambda i: (0, 0)),
            pl.BlockSpec((D_H, N_CLS), lambda i: (0, 0)),
            pl.BlockSpec((1, N_CLS), lambda i: (0, 0)),
        ],
        out_specs=pl.BlockSpec((br, N_CLS), lambda i: (i, 0)),
        out_shape=jax.ShapeDtypeStruct((N_PAD, N_CLS), jnp.float32),
    )


def kernel(x, edge_index, W1, b1, g1, be1, W2, b2, g2, be2, Wc, bc):
    src = edge_index[0]
    dst = edge_index[1]
    pad = E_PAD - E
    padv_src = N + (jnp.arange(pad, dtype=jnp.int32) % (N_PAD - N))
    padv_dst = N + ((jnp.arange(pad, dtype=jnp.int32) + 97) % (N_PAD - N))
    srcp = jnp.concatenate([src, padv_src]).reshape(-1, K)
    dstp = jnp.concatenate([dst, padv_dst]).reshape(-1, K)
    x_pad = jnp.pad(x, ((0, N_PAD - N), (0, 0)))

    s1 = g1 / jnp.sqrt(1.0 + BN_EPS)
    W1f = W1 * s1[None, :]
    b1f = (b1 * s1 + be1)[None, :]
    s2 = g2 / jnp.sqrt(1.0 + BN_EPS)
    W2f = W2 * s2[None, :]
    b2f = (b2 * s2 + be2)[None, :]

    agg1 = _sc_aggr_edgesplit()(x_pad, srcp, dstp)
    h1lo, h1hi = _tc_layer1(1024)(x_pad, agg1, agg1, W1f, b1f)
    agg2 = _sc_aggr_colsplit()(h1lo, h1hi, srcp, dstp)
    out = _tc_layer2(1024)(h1lo, h1hi, agg2, agg2, W2f, b2f, Wc, bc[None, :])
    return out[:N]


# speedup vs baseline: 3.1875x; 1.0111x over previous
import functools

import jax
import jax.numpy as jnp
from jax import lax
from jax.experimental import pallas as pl
from jax.experimental.pallas import tpu as pltpu
from jax.experimental.pallas import tpu_sc as plsc

N = 10000
E = 320000
D_IN = 128
D_H = 256
N_CLS = 2
BN_EPS = 1e-5

NC = 2
NS = 16
K = 128
N_PAD = 10240
CHUNKS = 160
E_PAD = NS * CHUNKS * K
GI = 40


def _zero_acc(rows0, acc, sid, rows_pt, dh):
    zero = jnp.zeros((16,), jnp.float32)

    def zrow(i, carry):
        for kk in range(dh // 16):
            rows0[i, pl.ds(kk * 16, 16)] = zero
        return carry

    lax.fori_loop(0, K, zrow, 0)
    rbase = sid * rows_pt
    for b in range(rows_pt // K):
        pltpu.sync_copy(rows0, acc.at[pl.ds(rbase + b * K, K)])


def _edge_pipeline(x_hbm, src_hbm, dst_hbm, tbase, n_groups,
                   src_v, dst_v, rows, gsems, ssems, acc):
    def group(g, carry):
        gb = tbase + g * GI
        pltpu.sync_copy(src_hbm.at[pl.ds(gb, GI)], src_v)
        pltpu.sync_copy(dst_hbm.at[pl.ds(gb, GI)], dst_v)
        gdesc = [None, None]
        gdesc[0] = pltpu.async_copy(x_hbm.at[src_v.at[0]], rows[0], gsems[0])
        for j in range(GI):
            b = j & 1
            nb = 1 - b
            if j + 1 < GI:
                gdesc[nb] = pltpu.async_copy(
                    x_hbm.at[src_v.at[j + 1]], rows[nb], gsems[nb])
            gdesc[b].wait()
            pltpu.sync_copy(rows[b], acc.at[dst_v.at[j]], add=True)
        return carry

    lax.fori_loop(0, n_groups, group, 0)


def _sc_aggr_edgesplit():
    rows_pt = N_PAD // NS
    ch = E_PAD // (NC * NS * K)
    mesh = plsc.VectorSubcoreMesh(
        core_axis_name="c", subcore_axis_name="s",
        num_cores=NC, num_subcores=NS)

    @functools.partial(
        pl.kernel,
        out_type=jax.ShapeDtypeStruct((2, N_PAD, D_IN), jnp.float32),
        mesh=mesh,
        scratch_types=[
            pltpu.VMEM((GI, K), jnp.int32),
            pltpu.VMEM((GI, K), jnp.int32),
            pltpu.VMEM((K, D_IN), jnp.float32),
            pltpu.VMEM((K, D_IN), jnp.float32),
            pltpu.VMEM_SHARED((N_PAD, D_IN), jnp.float32),
            pltpu.SemaphoreType.DMA,
            pltpu.SemaphoreType.DMA,
            pltpu.SemaphoreType.DMA,
            pltpu.SemaphoreType.DMA,
        ],
    )
    def aggr(x_hbm, src_hbm, dst_hbm, out,
             src_v, dst_v, rows0, rows1, acc, g0, g1, s0, s1):
        cid = lax.axis_index("c")
        sid = lax.axis_index("s")
        tbase = (cid * NS + sid) * ch
        _zero_acc(rows0, acc, sid, rows_pt, D_IN)
        plsc.subcore_barrier()
        _edge_pipeline(x_hbm, src_hbm, dst_hbm, tbase, ch // GI,
                       src_v, dst_v, [rows0, rows1], [g0, g1], [s0, s1], acc)
        plsc.subcore_barrier()
        rbase = sid * rows_pt
        pltpu.sync_copy(acc.at[pl.ds(rbase, rows_pt)],
                        out.at[cid, pl.ds(rbase, rows_pt)])

    return aggr


def _sc_aggr_colsplit():
    dh = 128
    rows_pt = N_PAD // NS
    mesh = plsc.VectorSubcoreMesh(
        core_axis_name="c", subcore_axis_name="s",
        num_cores=NC, num_subcores=NS)

    @functools.partial(
        pl.kernel,
        out_type=jax.ShapeDtypeStruct((2, N_PAD, dh), jnp.float32),
        mesh=mesh,
        scratch_types=[
            pltpu.VMEM((GI, K), jnp.int32),
            pltpu.VMEM((GI, K), jnp.int32),
            pltpu.VMEM((K, dh), jnp.float32),
            pltpu.VMEM((K, dh), jnp.float32),
            pltpu.VMEM_SHARED((N_PAD, dh), jnp.float32),
            pltpu.SemaphoreType.DMA,
            pltpu.SemaphoreType.DMA,
            pltpu.SemaphoreType.DMA,
            pltpu.SemaphoreType.DMA,
        ],
    )
    def aggr(xlo, xhi, src_hbm, dst_hbm, out,
             src_v, dst_v, rows0, rows1, acc, g0, g1, s0, s1):
        cid = lax.axis_index("c")
        sid = lax.axis_index("s")
        tbase = sid * CHUNKS
        _zero_acc(rows0, acc, sid, rows_pt, dh)
        plsc.subcore_barrier()

        @pl.when(cid == 0)
        def _():
            _edge_pipeline(xlo, src_hbm, dst_hbm, tbase, CHUNKS // GI,
                           src_v, dst_v, [rows0, rows1], [g0, g1], [s0, s1],
                           acc)

        @pl.when(cid == 1)
        def _():
            _edge_pipeline(xhi, src_hbm, dst_hbm, tbase, CHUNKS // GI,
                           src_v, dst_v, [rows0, rows1], [g0, g1], [s0, s1],
                           acc)

        plsc.subcore_barrier()
        rbase = sid * rows_pt
        pltpu.sync_copy(acc.at[pl.ds(rbase, rows_pt)],
                        out.at[cid, pl.ds(rbase, rows_pt)])

    return aggr


def _tc_layer1(br):
    grid = N // br

    def body(x_ref, agga_ref, aggb_ref, w_ref, b_ref, lo_ref, hi_ref):
        z = x_ref[...] + agga_ref[0] + aggb_ref[0]
        h = jnp.dot(z, w_ref[...], preferred_element_type=jnp.float32)
        h = jnp.maximum(h + b_ref[...], 0.0)
        lo_ref[...] = h[:, :128]
        hi_ref[...] = h[:, 128:]

    return pl.pallas_call(
        body,
        grid=(grid,),
        in_specs=[
            pl.BlockSpec((br, D_IN), lambda i: (i, 0)),
            pl.BlockSpec((1, br, D_IN), lambda i: (0, i, 0)),
            pl.BlockSpec((1, br, D_IN), lambda i: (1, i, 0)),
            pl.BlockSpec((D_IN, D_H), lambda i: (0, 0)),
            pl.BlockSpec((1, D_H), lambda i: (0, 0)),
        ],
        out_specs=[
            pl.BlockSpec((br, 128), lambda i: (i, 0)),
            pl.BlockSpec((br, 128), lambda i: (i, 0)),
        ],
        out_shape=[jax.ShapeDtypeStruct((N, 128), jnp.float32)] * 2,
    )


def _tc_layer2(br):
    grid = N // br

    def body(lo_ref, hi_ref, agglo_ref, agghi_ref, w_ref, b_ref, wc_ref, bc_ref,
             out_ref):
        h1 = jnp.concatenate([lo_ref[...], hi_ref[...]], axis=1)
        agg = jnp.concatenate([agglo_ref[0], agghi_ref[0]], axis=1)
        z = h1 + agg
        h = jnp.dot(z, w_ref[...], preferred_element_type=jnp.float32)
        h = jnp.maximum(h + b_ref[...], 0.0)
        out_ref[...] = (jnp.dot(h, wc_ref[...], preferred_element_type=jnp.float32)
                        + bc_ref[...])

    return pl.pallas_call(
        body,
        grid=(grid,),
        in_specs=[
            pl.BlockSpec((br, 128), lambda i: (i, 0)),
            pl.BlockSpec((br, 128), lambda i: (i, 0)),
            pl.BlockSpec((1, br, 128), lambda i: (0, i, 0)),
            pl.BlockSpec((1, br, 128), lambda i: (1, i, 0)),
            pl.BlockSpec((D_H, D_H), lambda i: (0, 0)),
            pl.BlockSpec((1, D_H), lambda i: (0, 0)),
            pl.BlockSpec((D_H, N_CLS), lambda i: (0, 0)),
            pl.BlockSpec((1, N_CLS), lambda i: (0, 0)),
        ],
        out_specs=pl.BlockSpec((br, N_CLS), lambda i: (i, 0)),
        out_shape=jax.ShapeDtypeStruct((N, N_CLS), jnp.float32),
    )


def kernel(x, edge_index, W1, b1, g1, be1, W2, b2, g2, be2, Wc, bc):
    src = edge_index[0]
    dst = edge_index[1]
    pad = E_PAD - E
    padv_src = jnp.arange(pad, dtype=jnp.int32) % N
    padv_dst = N + (jnp.arange(pad, dtype=jnp.int32) % (N_PAD - N))
    srcp = jnp.concatenate([src, padv_src]).reshape(-1, K)
    dstp = jnp.concatenate([dst, padv_dst]).reshape(-1, K)

    s1 = g1 / jnp.sqrt(1.0 + BN_EPS)
    W1f = W1 * s1[None, :]
    b1f = (b1 * s1 + be1)[None, :]
    s2 = g2 / jnp.sqrt(1.0 + BN_EPS)
    W2f = W2 * s2[None, :]
    b2f = (b2 * s2 + be2)[None, :]

    agg1 = _sc_aggr_edgesplit()(x, srcp, dstp)
    h1lo, h1hi = _tc_layer1(1000)(x, agg1, agg1, W1f, b1f)
    agg2 = _sc_aggr_colsplit()(h1lo, h1hi, srcp, dstp)
    return _tc_layer2(1000)(h1lo, h1hi, agg2, agg2, W2f, b2f, Wc, bc[None, :])


# speedup vs baseline: 3.2396x; 1.0164x over previous
import functools

import jax
import jax.numpy as jnp
from jax import lax
from jax.experimental import pallas as pl
from jax.experimental.pallas import tpu as pltpu
from jax.experimental.pallas import tpu_sc as plsc

N = 10000
E = 320000
D_IN = 128
D_H = 256
N_CLS = 2
BN_EPS = 1e-5

NC = 2
NS = 16
K = 128
N_PAD = 10240
CHUNKS = 160
E_PAD = NS * CHUNKS * K
GI = 40


def _zero_acc(rows0, acc, sid, rows_pt, dh):
    zero = jnp.zeros((16,), jnp.float32)

    def zrow(i, carry):
        for kk in range(dh // 16):
            rows0[i, pl.ds(kk * 16, 16)] = zero
        return carry

    lax.fori_loop(0, K, zrow, 0)
    rbase = sid * rows_pt
    for b in range(rows_pt // K):
        pltpu.sync_copy(rows0, acc.at[pl.ds(rbase + b * K, K)])


def _edge_pipeline(x_hbm, ei_hbm, tbase, n_groups,
                   src_v, dst_v, rows, gsems, ssems, acc):
    def group(g, carry):
        gb = tbase + g * GI
        pltpu.sync_copy(ei_hbm.at[0, pl.ds(gb, GI)], src_v)
        pltpu.sync_copy(ei_hbm.at[1, pl.ds(gb, GI)], dst_v)
        gdesc = [None, None]
        gdesc[0] = pltpu.async_copy(x_hbm.at[src_v.at[0]], rows[0], gsems[0])
        for j in range(GI):
            b = j & 1
            nb = 1 - b
            if j + 1 < GI:
                gdesc[nb] = pltpu.async_copy(
                    x_hbm.at[src_v.at[j + 1]], rows[nb], gsems[nb])
            gdesc[b].wait()
            pltpu.sync_copy(rows[b], acc.at[dst_v.at[j]], add=True)
        return carry

    lax.fori_loop(0, n_groups, group, 0)


def _sc_aggr_edgesplit():
    rows_pt = N_PAD // NS
    ch = E_PAD // (NC * NS * K)
    mesh = plsc.VectorSubcoreMesh(
        core_axis_name="c", subcore_axis_name="s",
        num_cores=NC, num_subcores=NS)

    @functools.partial(
        pl.kernel,
        out_type=jax.ShapeDtypeStruct((2, N_PAD, D_IN), jnp.float32),
        mesh=mesh,
        scratch_types=[
            pltpu.VMEM((GI, K), jnp.int32),
            pltpu.VMEM((GI, K), jnp.int32),
            pltpu.VMEM((K, D_IN), jnp.float32),
            pltpu.VMEM((K, D_IN), jnp.float32),
            pltpu.VMEM_SHARED((N_PAD, D_IN), jnp.float32),
            pltpu.SemaphoreType.DMA,
            pltpu.SemaphoreType.DMA,
            pltpu.SemaphoreType.DMA,
            pltpu.SemaphoreType.DMA,
        ],
    )
    def aggr(x_hbm, ei_hbm, out,
             src_v, dst_v, rows0, rows1, acc, g0, g1, s0, s1):
        cid = lax.axis_index("c")
        sid = lax.axis_index("s")
        tbase = (cid * NS + sid) * ch
        _zero_acc(rows0, acc, sid, rows_pt, D_IN)
        plsc.subcore_barrier()
        _edge_pipeline(x_hbm, ei_hbm, tbase, ch // GI,
                       src_v, dst_v, [rows0, rows1], [g0, g1], [s0, s1], acc)
        plsc.subcore_barrier()
        rbase = sid * rows_pt
        pltpu.sync_copy(acc.at[pl.ds(rbase, rows_pt)],
                        out.at[cid, pl.ds(rbase, rows_pt)])

    return aggr


def _sc_aggr_colsplit():
    dh = 128
    rows_pt = N_PAD // NS
    mesh = plsc.VectorSubcoreMesh(
        core_axis_name="c", subcore_axis_name="s",
        num_cores=NC, num_subcores=NS)

    @functools.partial(
        pl.kernel,
        out_type=jax.ShapeDtypeStruct((2, N_PAD, dh), jnp.float32),
        mesh=mesh,
        scratch_types=[
            pltpu.VMEM((GI, K), jnp.int32),
            pltpu.VMEM((GI, K), jnp.int32),
            pltpu.VMEM((K, dh), jnp.float32),
            pltpu.VMEM((K, dh), jnp.float32),
            pltpu.VMEM_SHARED((N_PAD, dh), jnp.float32),
            pltpu.SemaphoreType.DMA,
            pltpu.SemaphoreType.DMA,
            pltpu.SemaphoreType.DMA,
            pltpu.SemaphoreType.DMA,
        ],
    )
    def aggr(xlo, xhi, ei_hbm, out,
             src_v, dst_v, rows0, rows1, acc, g0, g1, s0, s1):
        cid = lax.axis_index("c")
        sid = lax.axis_index("s")
        tbase = sid * CHUNKS
        _zero_acc(rows0, acc, sid, rows_pt, dh)
        plsc.subcore_barrier()

        @pl.when(cid == 0)
        def _():
            _edge_pipeline(xlo, ei_hbm, tbase, CHUNKS // GI,
                           src_v, dst_v, [rows0, rows1], [g0, g1], [s0, s1],
                           acc)

        @pl.when(cid == 1)
        def _():
            _edge_pipeline(xhi, ei_hbm, tbase, CHUNKS // GI,
                           src_v, dst_v, [rows0, rows1], [g0, g1], [s0, s1],
                           acc)

        plsc.subcore_barrier()
        rbase = sid * rows_pt
        pltpu.sync_copy(acc.at[pl.ds(rbase, rows_pt)],
                        out.at[cid, pl.ds(rbase, rows_pt)])

    return aggr


def _tc_layer1(br):
    grid = N // br

    def body(x_ref, agga_ref, aggb_ref, w_ref, b_ref, lo_ref, hi_ref):
        z = x_ref[...] + agga_ref[0] + aggb_ref[0]
        h = jnp.dot(z, w_ref[...], preferred_element_type=jnp.float32)
        h = jnp.maximum(h + b_ref[...], 0.0)
        lo_ref[...] = h[:, :128]
        hi_ref[...] = h[:, 128:]

    return pl.pallas_call(
        body,
        grid=(grid,),
        in_specs=[
            pl.BlockSpec((br, D_IN), lambda i: (i, 0)),
            pl.BlockSpec((1, br, D_IN), lambda i: (0, i, 0)),
            pl.BlockSpec((1, br, D_IN), lambda i: (1, i, 0)),
            pl.BlockSpec((D_IN, D_H), lambda i: (0, 0)),
            pl.BlockSpec((1, D_H), lambda i: (0, 0)),
        ],
        out_specs=[
            pl.BlockSpec((br, 128), lambda i: (i, 0)),
            pl.BlockSpec((br, 128), lambda i: (i, 0)),
        ],
        out_shape=[jax.ShapeDtypeStruct((N, 128), jnp.float32)] * 2,
    )


def _tc_layer2(br):
    grid = N // br

    def body(lo_ref, hi_ref, agglo_ref, agghi_ref, w_ref, b_ref, wc_ref, bc_ref,
             out_ref):
        h1 = jnp.concatenate([lo_ref[...], hi_ref[...]], axis=1)
        agg = jnp.concatenate([agglo_ref[0], agghi_ref[0]], axis=1)
        z = h1 + agg
        h = jnp.dot(z, w_ref[...], preferred_element_type=jnp.float32)
        h = jnp.maximum(h + b_ref[...], 0.0)
        out_ref[...] = (jnp.dot(h, wc_ref[...], preferred_element_type=jnp.float32)
                        + bc_ref[...])

    return pl.pallas_call(
        body,
        grid=(grid,),
        in_specs=[
            pl.BlockSpec((br, 128), lambda i: (i, 0)),
            pl.BlockSpec((br, 128), lambda i: (i, 0)),
            pl.BlockSpec((1, br, 128), lambda i: (0, i, 0)),
            pl.BlockSpec((1, br, 128), lambda i: (1, i, 0)),
            pl.BlockSpec((D_H, D_H), lambda i: (0, 0)),
            pl.BlockSpec((1, D_H), lambda i: (0, 0)),
            pl.BlockSpec((D_H, N_CLS), lambda i: (0, 0)),
            pl.BlockSpec((1, N_CLS), lambda i: (0, 0)),
        ],
        out_specs=pl.BlockSpec((br, N_CLS), lambda i: (i, 0)),
        out_shape=jax.ShapeDtypeStruct((N, N_CLS), jnp.float32),
    )


def kernel(x, edge_index, W1, b1, g1, be1, W2, b2, g2, be2, Wc, bc):
    src = edge_index[0]
    dst = edge_index[1]
    pad = E_PAD - E
    padv_src = jnp.arange(pad, dtype=jnp.int32) % N
    padv_dst = N + (jnp.arange(pad, dtype=jnp.int32) % (N_PAD - N))
    eip = jnp.concatenate(
        [edge_index, jnp.stack([padv_src, padv_dst])], axis=1
    ).reshape(2, -1, K)

    s1 = g1 / jnp.sqrt(1.0 + BN_EPS)
    W1f = W1 * s1[None, :]
    b1f = (b1 * s1 + be1)[None, :]
    s2 = g2 / jnp.sqrt(1.0 + BN_EPS)
    W2f = W2 * s2[None, :]
    b2f = (b2 * s2 + be2)[None, :]

    agg1 = _sc_aggr_edgesplit()(x, eip)
    h1lo, h1hi = _tc_layer1(1000)(x, agg1, agg1, W1f, b1f)
    agg2 = _sc_aggr_colsplit()(h1lo, h1hi, eip)
    return _tc_layer2(1000)(h1lo, h1hi, agg2, agg2, W2f, b2f, Wc, bc[None, :])
